# Initial kernel scaffold; baseline (speedup 1.0000x reference)
import jax
import jax.numpy as jnp
from jax.experimental import pallas as pl


def kernel(lattices, form_table, lemma_table, tag_table, feats_table):
    raise NotImplementedError("write your pallas kernel here")


# speedup vs baseline: 9.2629x; 9.2629x over previous
import functools

import jax
import jax.numpy as jnp
from jax import lax
from jax.experimental import pallas as pl
from jax.experimental.pallas import tpu as pltpu
from jax.experimental.pallas import tpu_sc as plsc

L = 16
NC, NS = 2, 16
NW = NC * NS
CHUNK = 512

_FIELDS = (
    (0, 0, 32),
    (1, 32, 32),
    (2, 64, 16),
    (3, 80, 16),
    (3, 96, 16),
    (3, 112, 16),
    (3, 128, 16),
    (3, 144, 16),
)


def _sc_body(lat_hbm, form_hbm, lemma_hbm, tag_hbm, feats_hbm, out_hbm,
             lat_v, *rest):
    idx_vs, row_vs, sem = rest[:8], rest[8:16], rest[16]
    tables = (form_hbm, lemma_hbm, tag_hbm, feats_hbm)
    n_rows = out_hbm.shape[0]
    per_w = n_rows // NW
    n_chunks = per_w // CHUNK

    wid = lax.axis_index("s") * NC + lax.axis_index("c")
    lane = lax.iota(jnp.int32, L)
    pats = [lane * 8 + f for f in range(8)]

    def chunk_body(ci, carry):
        base = wid * per_w + ci * CHUNK
        pltpu.sync_copy(lat_hbm.at[pl.ds(base * 8, CHUNK * 8)], lat_v)

        def extract(j, c):
            off = j * (L * 8)
            for f in range(8):
                vals = plsc.load_gather(lat_v, [pats[f] + off])
                idx_vs[f][pl.ds(j * L, L)] = vals
            return c

        lax.fori_loop(0, CHUNK // L, extract, 0, unroll=2)

        handles = []
        for f, (t, _, _) in enumerate(_FIELDS):
            handles.append(
                pltpu.async_copy(tables[t].at[idx_vs[f]], row_vs[f], sem))
        for h in handles:
            h.wait()

        for f, (_, col, width) in enumerate(_FIELDS):
            pltpu.sync_copy(
                row_vs[f], out_hbm.at[pl.ds(base, CHUNK), pl.ds(col, width)])
        return carry

    lax.fori_loop(0, n_chunks, chunk_body, 0)


def kernel(lattices, form_table, lemma_table, tag_table, feats_table):
    b, t, a, m, _ = lattices.shape
    n = b * t * a * m
    lat_flat = lattices.reshape(n * 8)

    scratch_types = [pltpu.VMEM((CHUNK * 8,), jnp.int32)]
    scratch_types += [pltpu.VMEM((CHUNK,), jnp.int32) for _ in range(8)]
    scratch_types += [pltpu.VMEM((CHUNK, w), jnp.float32)
                      for (_, _, w) in _FIELDS]
    scratch_types.append(pltpu.SemaphoreType.DMA)

    mesh = plsc.VectorSubcoreMesh(core_axis_name="c", subcore_axis_name="s")
    out = pl.kernel(
        _sc_body,
        mesh=mesh,
        out_type=jax.ShapeDtypeStruct((n, 160), jnp.float32),
        scratch_types=scratch_types,
        compiler_params=pltpu.CompilerParams(
            use_tc_tiling_on_sc=False, needs_layout_passes=False),
    )(lat_flat, form_table, lemma_table, tag_table, feats_table)
    return out.reshape(b, t, a, m, 160)
